# Initial kernel scaffold; baseline (speedup 1.0000x reference)
import jax
import jax.numpy as jnp
from jax.experimental import pallas as pl


def kernel(features, w1, b1, w_obj, b_obj, w_box, b_box):
    raise NotImplementedError("write your pallas kernel here")


# speedup vs baseline: 8.8954x; 8.8954x over previous
import math

import numpy as np
import jax
import jax.numpy as jnp
from jax import lax
from jax.experimental import pallas as pl
from jax.experimental.pallas import tpu as pltpu

f32 = jnp.float32
bf16 = jnp.bfloat16

H = 50
W = 50
A = 3
PRE = 2000
POST = 1000
NMS_T = 0.7
IMG = 800.0
MIN_SIZE = 1.0
BBOX_CLIP = math.log(1000.0 / 16.0)
N7500 = H * W * A
NPAD = 7680
NCH = 60
NSORT = 2048
NBLK = 16
NOUT = 1024


def _anchor_arrays():
    size = 128.0
    ratios = np.array([0.5, 1.0, 2.0])
    h_ratios = np.sqrt(ratios)
    w_ratios = 1.0 / h_ratios
    ws = w_ratios * size
    hs = h_ratios * size
    base = np.stack([-ws / 2.0, -hs / 2.0, ws / 2.0, hs / 2.0], axis=1)
    sx = np.arange(W) * 16
    sy = np.arange(H) * 16
    yy, xx = np.meshgrid(sy, sx, indexing='ij')
    shifts = np.stack([xx.ravel(), yy.ravel(), xx.ravel(), yy.ravel()], axis=1)
    anchors = (shifts[:, None, :] + base[None, :, :]).reshape(-1, 4).astype(np.float32)
    wa = anchors[:, 2] - anchors[:, 0]
    ha = anchors[:, 3] - anchors[:, 1]
    cxa = anchors[:, 0] + np.float32(0.5) * wa
    cya = anchors[:, 1] + np.float32(0.5) * ha

    def padf(v):
        return np.pad(v, (0, NPAD - N7500), constant_values=1.0).reshape(NCH, 128)
    return padf(wa), padf(ha), padf(cxa), padf(cya)


_WA, _HA, _CXA, _CYA = _anchor_arrays()
_VMASK = (np.arange(NPAD) < N7500).astype(np.float32).reshape(NCH, 128)
_IDXG = np.arange(NPAD, dtype=np.float32).reshape(NCH, 128)


def _stage_a(x_ref, w_ref, b1_ref, whd_ref, bhd_ref, o_ref):
    X = x_ref[...].astype(f32)
    T = lax.dot_general(X, w_ref[...], (((0,), (0,)), ((), ())),
                        preferred_element_type=f32)
    T = jnp.maximum(T + b1_ref[...], 0.0)
    Hd = lax.dot_general(T, whd_ref[...], (((1,), (0,)), ((), ())),
                         preferred_element_type=f32)
    o_ref[...] = Hd + bhd_ref[...]


def _stage_b(obj_ref, d0_ref, d1_ref, d2_ref, d3_ref,
             wa_ref, ha_ref, cxa_ref, cya_ref, vm_ref, ix_ref,
             out_ref, scr_ref):
    o = obj_ref[...]
    s = 1.0 / (1.0 + jnp.exp(-o))
    s = jnp.where(vm_ref[...] > 0.0, s, -1.0)
    wa = wa_ref[...]
    ha = ha_ref[...]
    dx = d0_ref[...]
    dy = d1_ref[...]
    dw = jnp.minimum(d2_ref[...], BBOX_CLIP)
    dh = jnp.minimum(d3_ref[...], BBOX_CLIP)
    cx = dx * wa + cxa_ref[...]
    cy = dy * ha + cya_ref[...]
    pw = jnp.exp(dw) * wa
    ph = jnp.exp(dh) * ha
    x1 = jnp.clip(cx - 0.5 * pw, 0.0, IMG)
    y1 = jnp.clip(cy - 0.5 * ph, 0.0, IMG)
    x2 = jnp.clip(cx + 0.5 * pw, 0.0, IMG)
    y2 = jnp.clip(cy + 0.5 * ph, 0.0, IMG)
    vv = jnp.where(((x2 - x1) >= MIN_SIZE) & ((y2 - y1) >= MIN_SIZE)
                   & (s >= 0.0), 1.0, 0.0)
    idxg = ix_ref[...]
    scr_ref[0] = s
    scr_ref[7] = idxg

    si3 = s[:, :, None]
    ii3 = idxg[:, :, None]

    def jstep(jc, rank):
        sj = scr_ref[0, pl.ds(jc, 1), :].reshape(1, 1, 128)
        jx = scr_ref[7, pl.ds(jc, 1), :].reshape(1, 1, 128)
        gt = jnp.where(sj > si3, 1.0, 0.0)
        eq = jnp.where((sj == si3) & (jx < ii3), 1.0, 0.0)
        return rank + jnp.sum(gt + eq, axis=2)

    rank = lax.fori_loop(0, NCH, jstep, jnp.zeros((NCH, 128), f32))
    pos = jnp.minimum(rank, float(NSORT - 1))
    scr_ref[1] = x1
    scr_ref[2] = y1
    scr_ref[3] = x2
    scr_ref[4] = y2
    scr_ref[5] = vv
    scr_ref[6] = pos

    iota_p = lax.broadcasted_iota(jnp.int32, (1, NSORT), 1).astype(f32)

    def cstep(jc, acc):
        pc = jnp.transpose(scr_ref[6, pl.ds(jc, 1), :])
        onehot = pc == iota_p
        cols = []
        for c in range(6):
            v = jnp.transpose(scr_ref[c, pl.ds(jc, 1), :])
            cols.append(jnp.sum(jnp.where(onehot, v, 0.0), axis=0)[None, :])
        return acc + jnp.concatenate(cols, axis=0)

    out_ref[...] = lax.fori_loop(0, NCH, cstep, jnp.zeros((6, NSORT), f32))


def _stage_c(tb_ref, out_ref, nk_ref, sb_ref):
    s = tb_ref[0]
    x1 = tb_ref[1]
    y1 = tb_ref[2]
    x2 = tb_ref[3]
    y2 = tb_ref[4]
    vv = tb_ref[5]
    area = (x2 - x1) * (y2 - y1)
    lane = lax.broadcasted_iota(jnp.int32, (1, 128), 1).astype(f32)
    rowi = lax.broadcasted_iota(jnp.int32, (NBLK, 128), 0).astype(f32)
    lanei = lax.broadcasted_iota(jnp.int32, (NBLK, 128), 1).astype(f32)
    gidx = rowi * 128.0 + lanei
    keep = jnp.where(gidx < float(PRE), vv, 0.0)

    x1_3 = x1[None, :, :]
    y1_3 = y1[None, :, :]
    x2_3 = x2[None, :, :]
    y2_3 = y2[None, :, :]
    ar_3 = area[None, :, :]

    for b in range(NBLK):
        x1b = jnp.transpose(x1[b:b + 1, :])[:, :, None]
        y1b = jnp.transpose(y1[b:b + 1, :])[:, :, None]
        x2b = jnp.transpose(x2[b:b + 1, :])[:, :, None]
        y2b = jnp.transpose(y2[b:b + 1, :])[:, :, None]
        arb = jnp.transpose(area[b:b + 1, :])[:, :, None]
        lt_x = jnp.maximum(x1b, x1_3)
        lt_y = jnp.maximum(y1b, y1_3)
        rb_x = jnp.minimum(x2b, x2_3)
        rb_y = jnp.minimum(y2b, y2_3)
        wz = jnp.maximum(rb_x - lt_x, 0.0)
        hz = jnp.maximum(rb_y - lt_y, 0.0)
        inter = wz * hz
        iou = inter / (arb + ar_3 - inter + 1e-9)
        srow = jnp.where(iou > NMS_T, 1.0, 0.0)
        sb_ref[...] = srow[:, b, :]
        kb = keep[b:b + 1, :]

        def istep(i, kb):
            row = sb_ref[pl.ds(i, 1), :]
            fi = i.astype(f32)
            ki = jnp.sum(jnp.where(lane == fi, kb, 0.0))
            supp = jnp.where((lane > fi) & (row > 0.0), 1.0, 0.0)
            return kb * (1.0 - supp * ki)

        kb = lax.fori_loop(0, 128, istep, kb)
        kbT = jnp.transpose(kb)[:, :, None]
        sup_any = jnp.max(jnp.where((kbT > 0.0) & (srow > 0.0), 1.0, 0.0),
                          axis=0)
        later = gidx >= float((b + 1) * 128)
        keep = jnp.where(later & (sup_any > 0.0), 0.0, keep)
        keep = jnp.where(rowi == float(b), kb, keep)

    nk = jnp.sum(keep)
    nk_ref[...] = jnp.full((1, 128), nk, f32)

    iota_o = lax.broadcasted_iota(jnp.int32, (1, NOUT), 1).astype(f32)
    lane_le = (lax.broadcasted_iota(jnp.int32, (128, 128), 0)
               <= lax.broadcasted_iota(jnp.int32, (128, 128), 1))
    acc = jnp.zeros((5, NOUT), f32)
    offset = jnp.zeros((), f32)
    for b in range(NBLK):
        kc = keep[b:b + 1, :]
        kcT = jnp.transpose(kc)
        csum = jnp.sum(jnp.where(lane_le, kcT, 0.0), axis=0)[None, :]
        posc = offset + csum - 1.0
        offset = offset + jnp.sum(kc)
        idxf = jnp.where(kc > 0.0, jnp.minimum(posc, float(NOUT - 1)),
                         float(NOUT - 1))
        idxT = jnp.transpose(idxf)
        onehot = (idxT == iota_o) & (kcT > 0.0)
        rows = []
        for src in (x1, y1, x2, y2, s):
            v = jnp.transpose(src[b:b + 1, :])
            rows.append(jnp.sum(jnp.where(onehot, v, 0.0), axis=0)[None, :])
        acc = acc + jnp.concatenate(rows, axis=0)
    out_ref[...] = acc


def kernel(features, w1, b1, w_obj, b_obj, w_box, b_box):
    x = features[0]
    xp = jnp.pad(x, ((0, 0), (1, 1), (1, 1)))
    xf = jnp.pad(xp.reshape(256, 2704), ((0, 0), (0, 2)))
    xf = xf.astype(bf16)
    taps = []
    wrows = []
    for k in range(9):
        ky, kx = divmod(k, 3)
        off = ky * 52 + kx
        taps.append(lax.dynamic_slice(xf, (0, off), (256, 2600)))
        wrows.append(w1[:, :, ky, kx].T)
    X = jnp.concatenate(taps, axis=0)
    Wt = jnp.concatenate(wrows, axis=0)
    wo = w_obj[:, :, 0, 0].T.astype(bf16).astype(f32)
    wb = w_box[:, :, 0, 0].T.astype(bf16).astype(f32)
    whd = jnp.concatenate([wo, wb, jnp.zeros((256, 1), f32)], axis=1)
    bhd = jnp.concatenate([b_obj, b_box, jnp.zeros((1,), f32)])[None, :]

    Hd = pl.pallas_call(
        _stage_a,
        out_shape=jax.ShapeDtypeStruct((2600, 16), f32),
    )(X, Wt, b1[None, :], whd, bhd)

    Hv = Hd.reshape(50, 52, 16)[:, :50, :].reshape(2500, 16)
    obj = Hv[:, :3].reshape(N7500)
    dcol = Hv[:, 3:15].reshape(2500 * 3, 4)

    def padch(v):
        return jnp.pad(v, (0, NPAD - N7500)).reshape(NCH, 128)

    objp = padch(obj)
    dps = [padch(dcol[:, c]) for c in range(4)]

    tbl = pl.pallas_call(
        _stage_b,
        out_shape=jax.ShapeDtypeStruct((6, NSORT), f32),
        scratch_shapes=[pltpu.VMEM((8, NCH, 128), f32)],
    )(objp, dps[0], dps[1], dps[2], dps[3],
      jnp.asarray(_WA), jnp.asarray(_HA), jnp.asarray(_CXA), jnp.asarray(_CYA),
      jnp.asarray(_VMASK), jnp.asarray(_IDXG))

    comp, nk_arr = pl.pallas_call(
        _stage_c,
        out_shape=[jax.ShapeDtypeStruct((5, NOUT), f32),
                   jax.ShapeDtypeStruct((1, 128), f32)],
        scratch_shapes=[pltpu.VMEM((128, 128), f32)],
    )(tbl.reshape(6, NBLK, 128))

    nk = nk_arr[0, 0]
    rv = jnp.arange(POST, dtype=f32) < nk
    boxes = jnp.where(rv[:, None], comp[:4, :POST].T, 0.0)
    scores = jnp.where(rv, comp[4, :POST], 0.0)
    return boxes, scores
